# baseline (speedup 1.0000x reference)
import math

import jax
import jax.numpy as jnp
from jax import lax
from jax.experimental import pallas as pl
from jax.experimental.pallas import tpu as pltpu
from jax.experimental.pallas import tpu_sc as plsc

_F = 64
_B = 16384
_REG = 0.01

_INFO = plsc.get_sparse_core_info()
_NC = _INFO.num_cores
_NS = _INFO.num_subcores
_L = _INFO.num_lanes
_NW = _NC * _NS
_BPW = _B // _NW
_CHUNK = 128
_NCHUNK = _BPW // _CHUNK


def _sc_kernel_body(U_hbm, V_hbm, u_hbm, i_hbm, j_hbm,
                    pui_hbm, puj_hbm, sq_hbm,
                    uidx_v, iidx_v, jidx_v,
                    urows_v, irows_v, jrows_v,
                    pui_v, puj_v, sq_v, sem):
    wid = lax.axis_index("s") * _NC + lax.axis_index("c")
    base = wid * _BPW

    pltpu.sync_copy(u_hbm.at[pl.ds(base, _BPW)], uidx_v)
    pltpu.sync_copy(i_hbm.at[pl.ds(base, _BPW)], iidx_v)
    pltpu.sync_copy(j_hbm.at[pl.ds(base, _BPW)], jidx_v)

    handles = []
    for c in range(_NCHUNK):
        s = c * _CHUNK
        handles.append(pltpu.async_copy(
            U_hbm.at[uidx_v.at[pl.ds(s, _CHUNK)]],
            urows_v.at[pl.ds(s, _CHUNK)], sem))
        handles.append(pltpu.async_copy(
            V_hbm.at[iidx_v.at[pl.ds(s, _CHUNK)]],
            irows_v.at[pl.ds(s, _CHUNK)], sem))
        handles.append(pltpu.async_copy(
            V_hbm.at[jidx_v.at[pl.ds(s, _CHUNK)]],
            jrows_v.at[pl.ds(s, _CHUNK)], sem))
    for h in handles:
        h.wait()

    zero = jnp.zeros((_L,), jnp.float32)
    nk = _F // _L

    def row_body(r, sq_carry):
        s_u, s_vi, s_vj = sq_carry
        us = [urows_v[r, pl.ds(k * _L, _L)] for k in range(nk)]
        vis = [irows_v[r, pl.ds(k * _L, _L)] for k in range(nk)]
        vjs = [jrows_v[r, pl.ds(k * _L, _L)] for k in range(nk)]
        pui = us[0] * vis[0]
        puj = us[0] * vjs[0]
        for k in range(1, nk):
            pui = pui + us[k] * vis[k]
            puj = puj + us[k] * vjs[k]
        for k in range(nk):
            s_u = s_u + us[k] * us[k]
            s_vi = s_vi + vis[k] * vis[k]
            s_vj = s_vj + vjs[k] * vjs[k]
        pui_v[r, pl.ds(0, _L)] = pui
        puj_v[r, pl.ds(0, _L)] = puj
        return (s_u, s_vi, s_vj)

    s_u, s_vi, s_vj = lax.fori_loop(0, _BPW, row_body, (zero, zero, zero))

    sq_v[...] = s_u + s_vi + s_vj
    pltpu.sync_copy(pui_v, pui_hbm.at[pl.ds(base, _BPW)])
    pltpu.sync_copy(puj_v, puj_hbm.at[pl.ds(base, _BPW)])
    pltpu.sync_copy(sq_v, sq_hbm.at[pl.ds(wid * _L, _L)])


@jax.jit
def _sc_call(U, V, u, i, j):
    mesh = plsc.VectorSubcoreMesh(core_axis_name="c", subcore_axis_name="s")
    fn = pl.kernel(
        _sc_kernel_body,
        mesh=mesh,
        compiler_params=pltpu.CompilerParams(use_tc_tiling_on_sc=False),
        out_type=[
            jax.ShapeDtypeStruct((_B, _L), jnp.float32),
            jax.ShapeDtypeStruct((_B, _L), jnp.float32),
            jax.ShapeDtypeStruct((_NW * _L,), jnp.float32),
        ],
        scratch_types=[
            pltpu.VMEM((_BPW,), jnp.int32),
            pltpu.VMEM((_BPW,), jnp.int32),
            pltpu.VMEM((_BPW,), jnp.int32),
            pltpu.VMEM((_BPW, _F), jnp.float32),
            pltpu.VMEM((_BPW, _F), jnp.float32),
            pltpu.VMEM((_BPW, _F), jnp.float32),
            pltpu.VMEM((_BPW, _L), jnp.float32),
            pltpu.VMEM((_BPW, _L), jnp.float32),
            pltpu.VMEM((_L,), jnp.float32),
            pltpu.SemaphoreType.DMA,
        ],
    )
    return fn(U, V, u, i, j)


def _tc_body(pui_ref, puj_ref, sq_ref, yui_ref, yuj_ref, loss_ref):
    yui = jnp.sum(pui_ref[...], axis=1, keepdims=True)
    yuj = jnp.sum(puj_ref[...], axis=1, keepdims=True)
    yui_ref[...] = yui
    yuj_ref[...] = yuj
    d = yui - yuj
    sp = jnp.log1p(jnp.exp(-jnp.abs(d))) + jnp.maximum(-d, 0.0)
    reg = _REG * jnp.sum(sq_ref[...])
    loss_ref[0, 0] = reg + jnp.sum(sp) * (1.0 / math.log(2.0))


@jax.jit
def _tc_fold(pui, puj, sq):
    yui, yuj, loss = pl.pallas_call(
        _tc_body,
        out_shape=[
            jax.ShapeDtypeStruct((_B, 1), jnp.float32),
            jax.ShapeDtypeStruct((_B, 1), jnp.float32),
            jax.ShapeDtypeStruct((1, 1), jnp.float32),
        ],
        in_specs=[
            pl.BlockSpec(memory_space=pltpu.VMEM),
            pl.BlockSpec(memory_space=pltpu.VMEM),
            pl.BlockSpec(memory_space=pltpu.VMEM),
        ],
        out_specs=[
            pl.BlockSpec(memory_space=pltpu.VMEM),
            pl.BlockSpec(memory_space=pltpu.VMEM),
            pl.BlockSpec(memory_space=pltpu.SMEM),
        ],
    )(pui, puj, sq.reshape(4, 128))
    return yui.reshape(_B), yuj.reshape(_B), loss[0, 0]


def kernel(U, V, u, i, j):
    pui, puj, sq = _sc_call(U, V, u, i, j)
    return _tc_fold(pui, puj, sq)


# speedup vs baseline: 1.0222x; 1.0222x over previous
import math

import jax
import jax.numpy as jnp
from jax import lax
from jax.experimental import pallas as pl
from jax.experimental.pallas import tpu as pltpu
from jax.experimental.pallas import tpu_sc as plsc

_F = 64
_B = 16384
_REG = 0.01

_INFO = plsc.get_sparse_core_info()
_NC = _INFO.num_cores
_NS = _INFO.num_subcores
_L = _INFO.num_lanes
_NW = _NC * _NS
_BPW = _B // _NW
_CHUNK = 128
_NCHUNK = _BPW // _CHUNK
_RPL = 128 // _L


def _sc_kernel_body(U2_hbm, V2_hbm, u_hbm, i_hbm, j_hbm,
                    pui_hbm, puj_hbm, sq_hbm,
                    uidx_v, iidx_v, jidx_v,
                    uh_v, ih_v, jh_v,
                    ubuf, vibuf, vjbuf,
                    pui_v, puj_v, sq_v, sem0, sem1):
    wid = lax.axis_index("s") * _NC + lax.axis_index("c")
    base = wid * _BPW

    pltpu.sync_copy(u_hbm.at[pl.ds(base, _BPW)], uidx_v)
    pltpu.sync_copy(i_hbm.at[pl.ds(base, _BPW)], iidx_v)
    pltpu.sync_copy(j_hbm.at[pl.ds(base, _BPW)], jidx_v)

    def half_body(q, _):
        s = q * _L
        uh_v[pl.ds(s, _L)] = lax.shift_right_logical(uidx_v[pl.ds(s, _L)], 1)
        ih_v[pl.ds(s, _L)] = lax.shift_right_logical(iidx_v[pl.ds(s, _L)], 1)
        jh_v[pl.ds(s, _L)] = lax.shift_right_logical(jidx_v[pl.ds(s, _L)], 1)
        return 0
    lax.fori_loop(0, _BPW // _L, half_body, 0)

    sems = (sem0, sem1)

    def fire(c):
        pb = c % 2
        s = c * _CHUNK
        pltpu.async_copy(U2_hbm.at[uh_v.at[pl.ds(s, _CHUNK)]],
                         ubuf.at[pb], sems[pb])
        pltpu.async_copy(V2_hbm.at[ih_v.at[pl.ds(s, _CHUNK)]],
                         vibuf.at[pb], sems[pb])
        pltpu.async_copy(V2_hbm.at[jh_v.at[pl.ds(s, _CHUNK)]],
                         vjbuf.at[pb], sems[pb])

    def drain(c):
        pb = c % 2
        s = c * _CHUNK
        pltpu.make_async_copy(U2_hbm.at[uh_v.at[pl.ds(s, _CHUNK)]],
                              ubuf.at[pb], sems[pb]).wait()
        pltpu.make_async_copy(V2_hbm.at[ih_v.at[pl.ds(s, _CHUNK)]],
                              vibuf.at[pb], sems[pb]).wait()
        pltpu.make_async_copy(V2_hbm.at[jh_v.at[pl.ds(s, _CHUNK)]],
                              vjbuf.at[pb], sems[pb]).wait()

    zero = jnp.zeros((_L,), jnp.float32)
    nk = _F // _L

    fire(0)
    sq_acc = (zero, zero, zero)
    for c in range(_NCHUNK):
        drain(c)
        if c + 1 < _NCHUNK:
            fire(c + 1)
        pb = c % 2
        ub, vib, vjb = ubuf.at[pb], vibuf.at[pb], vjbuf.at[pb]

        def grp_body(q, carry, c=c, ub=ub, vib=vib, vjb=vjb):
            s_u, s_vi, s_vj = carry
            g0 = c * _CHUNK + q * _L
            uoffs = (uidx_v[pl.ds(g0, _L)] & 1) * _F
            ioffs = (iidx_v[pl.ds(g0, _L)] & 1) * _F
            joffs = (jidx_v[pl.ds(g0, _L)] & 1) * _F
            for rr in range(_L):
                r = q * _L + rr
                uoff = uoffs[rr]
                ioff = ioffs[rr]
                joff = joffs[rr]
                us = [ub[r, pl.ds(uoff + k * _L, _L)] for k in range(nk)]
                vis = [vib[r, pl.ds(ioff + k * _L, _L)] for k in range(nk)]
                vjs = [vjb[r, pl.ds(joff + k * _L, _L)] for k in range(nk)]
                pui = us[0] * vis[0]
                puj = us[0] * vjs[0]
                for k in range(1, nk):
                    pui = pui + us[k] * vis[k]
                    puj = puj + us[k] * vjs[k]
                for k in range(nk):
                    s_u = s_u + us[k] * us[k]
                    s_vi = s_vi + vis[k] * vis[k]
                    s_vj = s_vj + vjs[k] * vjs[k]
                g = g0 + rr
                line = g // _RPL
                lane = (g % _RPL) * _L
                pui_v[line, pl.ds(lane, _L)] = pui
                puj_v[line, pl.ds(lane, _L)] = puj
            return (s_u, s_vi, s_vj)

        sq_acc = lax.fori_loop(0, _CHUNK // _L, grp_body, sq_acc)

    s_u, s_vi, s_vj = sq_acc
    sq_v[...] = s_u + s_vi + s_vj
    pltpu.sync_copy(pui_v, pui_hbm.at[pl.ds(wid * (_BPW // _RPL),
                                            _BPW // _RPL)])
    pltpu.sync_copy(puj_v, puj_hbm.at[pl.ds(wid * (_BPW // _RPL),
                                            _BPW // _RPL)])
    pltpu.sync_copy(sq_v, sq_hbm.at[pl.ds(wid * _L, _L)])


@jax.jit
def _sc_call(U2, V2, u, i, j):
    mesh = plsc.VectorSubcoreMesh(core_axis_name="c", subcore_axis_name="s")
    fn = pl.kernel(
        _sc_kernel_body,
        mesh=mesh,
        out_type=[
            jax.ShapeDtypeStruct((_B // _RPL, 128), jnp.float32),
            jax.ShapeDtypeStruct((_B // _RPL, 128), jnp.float32),
            jax.ShapeDtypeStruct((_NW * _L,), jnp.float32),
        ],
        scratch_types=[
            pltpu.VMEM((_BPW,), jnp.int32),
            pltpu.VMEM((_BPW,), jnp.int32),
            pltpu.VMEM((_BPW,), jnp.int32),
            pltpu.VMEM((_BPW,), jnp.int32),
            pltpu.VMEM((_BPW,), jnp.int32),
            pltpu.VMEM((_BPW,), jnp.int32),
            pltpu.VMEM((2, _CHUNK, 128), jnp.float32),
            pltpu.VMEM((2, _CHUNK, 128), jnp.float32),
            pltpu.VMEM((2, _CHUNK, 128), jnp.float32),
            pltpu.VMEM((_BPW // _RPL, 128), jnp.float32),
            pltpu.VMEM((_BPW // _RPL, 128), jnp.float32),
            pltpu.VMEM((_L,), jnp.float32),
            pltpu.SemaphoreType.DMA,
            pltpu.SemaphoreType.DMA,
        ],
    )
    return fn(U2, V2, u, i, j)


def _tc_body(pui_ref, puj_ref, sq_ref, yui_ref, yuj_ref, loss_ref):
    ri = lax.broadcasted_iota(jnp.int32, (128, _RPL), 0)
    ci = lax.broadcasted_iota(jnp.int32, (128, _RPL), 1)
    fold = (ri // _L == ci).astype(jnp.float32)
    yui = jax.lax.dot_general(pui_ref[...], fold, (((1,), (0,)), ((), ())),
                              preferred_element_type=jnp.float32)
    yuj = jax.lax.dot_general(puj_ref[...], fold, (((1,), (0,)), ((), ())),
                              preferred_element_type=jnp.float32)
    yui_ref[...] = yui
    yuj_ref[...] = yuj
    d = yui - yuj
    sp = jnp.log1p(jnp.exp(-jnp.abs(d))) + jnp.maximum(-d, 0.0)
    reg = _REG * jnp.sum(sq_ref[...])
    loss_ref[0, 0] = reg + jnp.sum(sp) * (1.0 / math.log(2.0))


@jax.jit
def _tc_fold(pui, puj, sq):
    yui8, yuj8, loss = pl.pallas_call(
        _tc_body,
        out_shape=[
            jax.ShapeDtypeStruct((_B // _RPL, _RPL), jnp.float32),
            jax.ShapeDtypeStruct((_B // _RPL, _RPL), jnp.float32),
            jax.ShapeDtypeStruct((1, 1), jnp.float32),
        ],
        in_specs=[
            pl.BlockSpec(memory_space=pltpu.VMEM),
            pl.BlockSpec(memory_space=pltpu.VMEM),
            pl.BlockSpec(memory_space=pltpu.VMEM),
        ],
        out_specs=[
            pl.BlockSpec(memory_space=pltpu.VMEM),
            pl.BlockSpec(memory_space=pltpu.VMEM),
            pl.BlockSpec(memory_space=pltpu.SMEM),
        ],
    )(pui, puj, sq.reshape(4, 128))
    return yui8.reshape(_B), yuj8.reshape(_B), loss[0, 0]


def kernel(U, V, u, i, j):
    U2 = U.reshape(1000000 // 2, 2 * _F)
    V2 = V.reshape(1000000 // 2, 2 * _F)
    pui, puj, sq = _sc_call(U2, V2, u, i, j)
    return _tc_fold(pui, puj, sq)


# speedup vs baseline: 1.0913x; 1.0676x over previous
import math

import jax
import jax.numpy as jnp
from jax import lax
from jax.experimental import pallas as pl
from jax.experimental.pallas import tpu as pltpu
from jax.experimental.pallas import tpu_sc as plsc

_F = 64
_B = 16384
_REG = 0.01

_INFO = plsc.get_sparse_core_info()
_NC = _INFO.num_cores
_NS = _INFO.num_subcores
_L = _INFO.num_lanes
_NW = _NC * _NS
_BPW = _B // _NW
_CHUNK = 128
_NCHUNK = _BPW // _CHUNK
_RPL = 128 // _L


def _sc_kernel_body(U2_hbm, V2_hbm, u_hbm, i_hbm, j_hbm,
                    pui_hbm, puj_hbm, sq_hbm,
                    uidx_v, iidx_v, jidx_v,
                    ubuf, vibuf, vjbuf,
                    pui_v, puj_v, sq_v, sem0, sem1):
    wid = lax.axis_index("s") * _NC + lax.axis_index("c")
    base = wid * _BPW

    pltpu.sync_copy(u_hbm.at[pl.ds(base, _BPW)], uidx_v)
    pltpu.sync_copy(i_hbm.at[pl.ds(base, _BPW)], iidx_v)
    pltpu.sync_copy(j_hbm.at[pl.ds(base, _BPW)], jidx_v)

    sems = (sem0, sem1)

    def fire(c):
        pb = c % 2
        s = c * _CHUNK
        pltpu.async_copy(U2_hbm.at[uidx_v.at[pl.ds(s, _CHUNK)]],
                         ubuf.at[pb], sems[pb])
        pltpu.async_copy(V2_hbm.at[iidx_v.at[pl.ds(s, _CHUNK)]],
                         vibuf.at[pb], sems[pb])
        pltpu.async_copy(V2_hbm.at[jidx_v.at[pl.ds(s, _CHUNK)]],
                         vjbuf.at[pb], sems[pb])

    def drain(c):
        pb = c % 2
        s = c * _CHUNK
        pltpu.make_async_copy(U2_hbm.at[uidx_v.at[pl.ds(s, _CHUNK)]],
                              ubuf.at[pb], sems[pb]).wait()
        pltpu.make_async_copy(V2_hbm.at[iidx_v.at[pl.ds(s, _CHUNK)]],
                              vibuf.at[pb], sems[pb]).wait()
        pltpu.make_async_copy(V2_hbm.at[jidx_v.at[pl.ds(s, _CHUNK)]],
                              vjbuf.at[pb], sems[pb]).wait()

    zero = jnp.zeros((_L,), jnp.float32)
    nk = _F // _L

    fire(0)
    sq_acc = (zero, zero, zero)
    for c in range(_NCHUNK):
        drain(c)
        if c + 1 < _NCHUNK:
            fire(c + 1)
        pb = c % 2
        ub, vib, vjb = ubuf.at[pb], vibuf.at[pb], vjbuf.at[pb]

        def grp_body(q, carry, c=c, ub=ub, vib=vib, vjb=vjb):
            s_u, s_vi, s_vj = carry
            for rr in range(_L):
                r = q * _L + rr
                us = [ub[r, pl.ds(k * _L, _L)] for k in range(nk)]
                vis = [vib[r, pl.ds(k * _L, _L)] for k in range(nk)]
                vjs = [vjb[r, pl.ds(k * _L, _L)] for k in range(nk)]
                pui = us[0] * vis[0]
                puj = us[0] * vjs[0]
                for k in range(1, nk):
                    pui = pui + us[k] * vis[k]
                    puj = puj + us[k] * vjs[k]
                for k in range(nk):
                    s_u = s_u + us[k] * us[k]
                    s_vi = s_vi + vis[k] * vis[k]
                    s_vj = s_vj + vjs[k] * vjs[k]
                g = c * _CHUNK + q * _L + rr
                line = g // _RPL
                lane = (g % _RPL) * _L
                pui_v[line, pl.ds(lane, _L)] = pui
                puj_v[line, pl.ds(lane, _L)] = puj
            return (s_u, s_vi, s_vj)

        sq_acc = lax.fori_loop(0, _CHUNK // _L, grp_body, sq_acc)

    s_u, s_vi, s_vj = sq_acc
    sq_v[...] = s_u + s_vi + s_vj
    pltpu.sync_copy(pui_v, pui_hbm.at[pl.ds(wid * (_BPW // _RPL),
                                            _BPW // _RPL)])
    pltpu.sync_copy(puj_v, puj_hbm.at[pl.ds(wid * (_BPW // _RPL),
                                            _BPW // _RPL)])
    pltpu.sync_copy(sq_v, sq_hbm.at[pl.ds(wid * _L, _L)])


@jax.jit
def _sc_call(U2, V2, u, i, j):
    mesh = plsc.VectorSubcoreMesh(core_axis_name="c", subcore_axis_name="s")
    fn = pl.kernel(
        _sc_kernel_body,
        mesh=mesh,
        out_type=[
            jax.ShapeDtypeStruct((_B // _RPL, 128), jnp.float32),
            jax.ShapeDtypeStruct((_B // _RPL, 128), jnp.float32),
            jax.ShapeDtypeStruct((_NW * _L,), jnp.float32),
        ],
        scratch_types=[
            pltpu.VMEM((_BPW,), jnp.int32),
            pltpu.VMEM((_BPW,), jnp.int32),
            pltpu.VMEM((_BPW,), jnp.int32),
            pltpu.VMEM((2, _CHUNK, 128), jnp.float32),
            pltpu.VMEM((2, _CHUNK, 128), jnp.float32),
            pltpu.VMEM((2, _CHUNK, 128), jnp.float32),
            pltpu.VMEM((_BPW // _RPL, 128), jnp.float32),
            pltpu.VMEM((_BPW // _RPL, 128), jnp.float32),
            pltpu.VMEM((_L,), jnp.float32),
            pltpu.SemaphoreType.DMA,
            pltpu.SemaphoreType.DMA,
        ],
    )
    return fn(U2, V2, u, i, j)


def _tc_body(pui_ref, puj_ref, sq_ref, yui_ref, yuj_ref, loss_ref):
    ri = lax.broadcasted_iota(jnp.int32, (128, _RPL), 0)
    ci = lax.broadcasted_iota(jnp.int32, (128, _RPL), 1)
    fold = (ri // _L == ci).astype(jnp.float32)
    yui = jax.lax.dot_general(pui_ref[...], fold, (((1,), (0,)), ((), ())),
                              preferred_element_type=jnp.float32)
    yuj = jax.lax.dot_general(puj_ref[...], fold, (((1,), (0,)), ((), ())),
                              preferred_element_type=jnp.float32)
    yui_ref[...] = yui
    yuj_ref[...] = yuj
    d = yui - yuj
    sp = jnp.log1p(jnp.exp(-jnp.abs(d))) + jnp.maximum(-d, 0.0)
    reg = _REG * jnp.sum(sq_ref[...])
    loss_ref[0, 0] = reg + jnp.sum(sp) * (1.0 / math.log(2.0))


@jax.jit
def _tc_fold(pui, puj, sq):
    yui8, yuj8, loss = pl.pallas_call(
        _tc_body,
        out_shape=[
            jax.ShapeDtypeStruct((_B // _RPL, _RPL), jnp.float32),
            jax.ShapeDtypeStruct((_B // _RPL, _RPL), jnp.float32),
            jax.ShapeDtypeStruct((1, 1), jnp.float32),
        ],
        in_specs=[
            pl.BlockSpec(memory_space=pltpu.VMEM),
            pl.BlockSpec(memory_space=pltpu.VMEM),
            pl.BlockSpec(memory_space=pltpu.VMEM),
        ],
        out_specs=[
            pl.BlockSpec(memory_space=pltpu.VMEM),
            pl.BlockSpec(memory_space=pltpu.VMEM),
            pl.BlockSpec(memory_space=pltpu.SMEM),
        ],
    )(pui, puj, sq.reshape(4, 128))
    return yui8.reshape(_B), yuj8.reshape(_B), loss[0, 0]


def kernel(U, V, u, i, j):
    U2 = jnp.pad(U, ((0, 0), (0, 128 - _F)))
    V2 = jnp.pad(V, ((0, 0), (0, 128 - _F)))
    pui, puj, sq = _sc_call(U2, V2, u, i, j)
    return _tc_fold(pui, puj, sq)
